# baseline (speedup 1.0000x reference)
import jax
import jax.numpy as jnp
from jax import lax
from jax.experimental import pallas as pl
from jax.experimental.pallas import tpu as pltpu
from jax.experimental.pallas import tpu_sc as plsc

B = 16384
D = 64
L = 16
NC = 2
NS = 16
NW = NC * NS
BPW = B // NW


def _mf_kernel(uid_hbm, iid_hbm, uemb_hbm, iemb_hbm, ub_hbm, ib_hbm, out_hbm,
               uid_v, iid_v, urows_v, irows_v, ub_v, ib_v, out_v, sem):
    wid = lax.axis_index("s") * NC + lax.axis_index("c")
    base = wid * BPW
    pltpu.sync_copy(uid_hbm.at[pl.ds(base, BPW)], uid_v)
    pltpu.sync_copy(iid_hbm.at[pl.ds(base, BPW)], iid_v)

    cps = [pltpu.async_copy(ub_hbm.at[uid_v], ub_v, sem),
           pltpu.async_copy(ib_hbm.at[iid_v], ib_v, sem)]
    for e in range(D):
        cps.append(pltpu.async_copy(uemb_hbm.at[e].at[uid_v],
                                    urows_v.at[e], sem))
        cps.append(pltpu.async_copy(iemb_hbm.at[e].at[iid_v],
                                    irows_v.at[e], sem))
    for cp in cps:
        cp.wait()

    @pl.loop(0, BPW, step=L)
    def _(i):
        acc = ub_v[pl.ds(i, L)] + ib_v[pl.ds(i, L)]
        for e in range(D):
            acc = acc + urows_v[e, pl.ds(i, L)] * irows_v[e, pl.ds(i, L)]
        out_v[pl.ds(i, L)] = acc

    pltpu.sync_copy(out_v, out_hbm.at[pl.ds(base, BPW)])


@jax.jit
def _mf(user_ids, item_ids, user_emb, item_emb, user_biases, item_biases):
    zu = user_emb.T
    zi = item_emb.T
    user_biases = user_biases.reshape(-1)
    item_biases = item_biases.reshape(-1)
    mesh = plsc.VectorSubcoreMesh(core_axis_name="c", subcore_axis_name="s")
    kfn = pl.kernel(
        _mf_kernel,
        mesh=mesh,
        compiler_params=pltpu.CompilerParams(
            needs_layout_passes=False, use_tc_tiling_on_sc=False),
        out_type=jax.ShapeDtypeStruct((B,), jnp.float32),
        scratch_types=[
            pltpu.VMEM((BPW,), jnp.int32),
            pltpu.VMEM((BPW,), jnp.int32),
            pltpu.VMEM((D, BPW), jnp.float32),
            pltpu.VMEM((D, BPW), jnp.float32),
            pltpu.VMEM((BPW,), jnp.float32),
            pltpu.VMEM((BPW,), jnp.float32),
            pltpu.VMEM((BPW,), jnp.float32),
            pltpu.SemaphoreType.DMA,
        ],
    )
    return kfn(user_ids, item_ids, zu, zi, user_biases, item_biases)


def kernel(user_ids, item_ids, user_emb, item_emb, user_biases, item_biases):
    return _mf(user_ids.astype(jnp.int32), item_ids.astype(jnp.int32),
               user_emb, item_emb, user_biases, item_biases)


# speedup vs baseline: 8.8204x; 8.8204x over previous
import jax
import jax.numpy as jnp
from jax import lax
from jax.experimental import pallas as pl
from jax.experimental.pallas import tpu as pltpu
from jax.experimental.pallas import tpu_sc as plsc

B = 16384
D = 64
L = 16
NC = 2
NS = 16
NW = NC * NS
BPW = B // NW
N_ROWS = 1000000
ZROWS = N_ROWS // 2
BROWS = 7832
CHUNK = 64


def _mf_kernel(uid_hbm, iid_hbm, zu_hbm, zi_hbm, ub_hbm, ib_hbm, out_hbm,
               uid_v, iid_v, uzr_v, izr_v, ubr_v, ibr_v,
               urows_v, irows_v, ubrow_v, ibrow_v, out_v, sem):
    wid = lax.axis_index("s") * NC + lax.axis_index("c")
    base = wid * BPW
    pltpu.sync_copy(uid_hbm.at[pl.ds(base, BPW)], uid_v)
    pltpu.sync_copy(iid_hbm.at[pl.ds(base, BPW)], iid_v)

    @pl.loop(0, BPW, step=L)
    def _(i):
        u = uid_v[pl.ds(i, L)]
        t = iid_v[pl.ds(i, L)]
        uzr_v[pl.ds(i, L)] = u >> 1
        izr_v[pl.ds(i, L)] = t >> 1
        ubr_v[pl.ds(i, L)] = u >> 7
        ibr_v[pl.ds(i, L)] = t >> 7

    iota = lax.iota(jnp.int32, L)
    one = jnp.full((L,), 1, jnp.int32)
    m127 = jnp.full((L,), 127, jnp.int32)
    six = jnp.full((L,), 6, jnp.int32)

    for c in range(BPW // CHUNK):
        cb = c * CHUNK
        cps = [pltpu.async_copy(zu_hbm.at[uzr_v.at[pl.ds(cb, CHUNK)]],
                                urows_v, sem),
               pltpu.async_copy(zi_hbm.at[izr_v.at[pl.ds(cb, CHUNK)]],
                                irows_v, sem),
               pltpu.async_copy(ub_hbm.at[ubr_v.at[pl.ds(cb, CHUNK)]],
                                ubrow_v, sem),
               pltpu.async_copy(ib_hbm.at[ibr_v.at[pl.ds(cb, CHUNK)]],
                                ibrow_v, sem)]
        for cp in cps:
            cp.wait()

        @pl.loop(0, CHUNK, step=L)
        def _(rb):
            row_idx = rb + iota
            u = uid_v[pl.ds(cb + rb, L)]
            t = iid_v[pl.ds(cb + rb, L)]
            uoff = (u & one) << six
            ioff = (t & one) << six
            acc = (plsc.load_gather(ubrow_v, [row_idx, u & m127])
                   + plsc.load_gather(ibrow_v, [row_idx, t & m127]))
            for k in range(D):
                ck = jnp.full((L,), k, jnp.int32)
                acc = acc + (plsc.load_gather(urows_v, [row_idx, uoff + ck])
                             * plsc.load_gather(irows_v, [row_idx, ioff + ck]))
            out_v[pl.ds(cb + rb, L)] = acc

    pltpu.sync_copy(out_v, out_hbm.at[pl.ds(base, BPW)])


@jax.jit
def _mf(user_ids, item_ids, user_emb, item_emb, user_biases, item_biases):
    zu = user_emb.reshape(ZROWS, 2 * D)
    zi = item_emb.reshape(ZROWS, 2 * D)
    ub = jnp.pad(user_biases.reshape(-1),
                 (0, BROWS * 128 - N_ROWS)).reshape(BROWS, 128)
    ib = jnp.pad(item_biases.reshape(-1),
                 (0, BROWS * 128 - N_ROWS)).reshape(BROWS, 128)
    mesh = plsc.VectorSubcoreMesh(core_axis_name="c", subcore_axis_name="s")
    kfn = pl.kernel(
        _mf_kernel,
        mesh=mesh,
        compiler_params=pltpu.CompilerParams(needs_layout_passes=False),
        out_type=jax.ShapeDtypeStruct((B,), jnp.float32),
        scratch_types=[
            pltpu.VMEM((BPW,), jnp.int32),
            pltpu.VMEM((BPW,), jnp.int32),
            pltpu.VMEM((BPW,), jnp.int32),
            pltpu.VMEM((BPW,), jnp.int32),
            pltpu.VMEM((BPW,), jnp.int32),
            pltpu.VMEM((BPW,), jnp.int32),
            pltpu.VMEM((CHUNK, 2 * D), jnp.float32),
            pltpu.VMEM((CHUNK, 2 * D), jnp.float32),
            pltpu.VMEM((CHUNK, 128), jnp.float32),
            pltpu.VMEM((CHUNK, 128), jnp.float32),
            pltpu.VMEM((BPW,), jnp.float32),
            pltpu.SemaphoreType.DMA,
        ],
    )
    return kfn(user_ids, item_ids, zu, zi, ub, ib)


def kernel(user_ids, item_ids, user_emb, item_emb, user_biases, item_biases):
    return _mf(user_ids.astype(jnp.int32), item_ids.astype(jnp.int32),
               user_emb, item_emb, user_biases, item_biases)
